# baseline (speedup 1.0000x reference)
import functools

import jax
import jax.numpy as jnp
from jax.experimental import pallas as pl
from jax.experimental.pallas import tpu as pltpu
from jax.experimental.pallas import tpu_sc as plsc

T, D, H, E, TOPK = 256, 1024, 512, 64, 2
S = T * TOPK
TILE = 32
NT = 78
NTE = 80
ROWS = 2560
F32 = jnp.float32


def _routing_body(eflat_ref, pflat_ref, pos_ref, row_tok_ref, prob2d_ref,
                  te_ref):
    eflat = eflat_ref[...]
    pflat = pflat_ref[...]
    iota_e = jax.lax.broadcasted_iota(jnp.int32, (1, E), 1)
    onehot = (eflat == iota_e).astype(F32)
    r_i = jax.lax.broadcasted_iota(jnp.int32, (S, S), 0)
    c_i = jax.lax.broadcasted_iota(jnp.int32, (S, S), 1)
    lts = (r_i >= c_i).astype(F32)
    cum = jnp.dot(lts, onehot, preferred_element_type=F32)
    rank = jnp.sum(onehot * (cum - 1.0), axis=1, keepdims=True)
    counts = jnp.sum(onehot, axis=0, keepdims=True)
    padded = jnp.floor((counts + (TILE - 1)) / TILE) * TILE
    r64 = jax.lax.broadcasted_iota(jnp.int32, (E, E), 0)
    c64 = jax.lax.broadcasted_iota(jnp.int32, (E, E), 1)
    mstrict = (r64 < c64).astype(F32)
    start = jnp.dot(padded, mstrict, preferred_element_type=F32, precision=jax.lax.Precision.HIGHEST)
    pos_f = jnp.sum(onehot * start, axis=1, keepdims=True) + rank
    eye = (r_i == c_i).astype(F32)
    pos_t = jnp.dot(jnp.ones((1, S), F32), eye * pos_f,
                    preferred_element_type=F32, precision=jax.lax.Precision.HIGHEST)
    rr = jax.lax.broadcasted_iota(jnp.int32, (ROWS, 1), 0).astype(F32)
    perm = (rr == pos_t).astype(F32)
    tok = (jax.lax.broadcasted_iota(jnp.int32, (S, 1), 0) // TOPK).astype(F32)
    rhs = jnp.concatenate([tok, pflat, jnp.ones((S, 1), F32)], axis=1)
    rowinfo = jnp.dot(perm, rhs, preferred_element_type=F32,
                      precision=jax.lax.Precision.HIGHEST)
    row_tok = rowinfo[:, 0:1]
    row_prob = rowinfo[:, 1:2]
    valid = rowinfo[:, 2:3]
    rr_mod = rr - jnp.floor(rr / T) * T
    row_tok = jnp.where(valid > 0.5, row_tok, rr_mod)
    ti = jax.lax.broadcasted_iota(jnp.int32, (NTE, 1), 0).astype(F32) * TILE
    te_hit = ((ti >= start) & (ti < start + padded)).astype(F32)
    tile_expert = jnp.sum(te_hit * iota_e.astype(F32), axis=1, keepdims=True)
    tile_hit = jnp.sum(te_hit, axis=1, keepdims=True)
    tile_expert = jnp.where(tile_hit > 0.5, tile_expert,
                            jnp.max(tile_expert))

    pos_ref[...] = pos_f.astype(jnp.int32)
    row_tok_ref[...] = row_tok.astype(jnp.int32)
    prob2d_ref[...] = jnp.broadcast_to(row_prob, (ROWS, 128))
    te_ref[...] = tile_expert.astype(jnp.int32)


def _routing(eflat, pflat):
    return pl.pallas_call(
        _routing_body,
        out_shape=(
            jax.ShapeDtypeStruct((S, 1), jnp.int32),
            jax.ShapeDtypeStruct((ROWS, 1), jnp.int32),
            jax.ShapeDtypeStruct((ROWS, 128), F32),
            jax.ShapeDtypeStruct((NTE, 1), jnp.int32),
        ),
    )(eflat, pflat)


NT2 = NTE // 2


def _gemm_body(te_ref, rt_ref, x_ref, prob_ref, wup_a_ref, wdn_a_ref,
               wup_b_ref, wdn_b_ref, out_ref, xg_ref):
    i = pl.program_id(0)
    for r in range(2 * TILE):
        tok = rt_ref[i * 2 * TILE + r]
        xg_ref[pl.ds(r, 1), :] = x_ref[pl.ds(tok, 1), :]
    c = 0.7071067811865476
    xa = xg_ref[pl.ds(0, TILE), :]
    hg_a = jnp.dot(xa, wup_a_ref[0], preferred_element_type=F32)
    h_a, g_a = hg_a[:, :H], hg_a[:, H:]
    act_a = (0.5 * h_a * (1.0 + jax.lax.erf(h_a * c))) * (g_a + 1.0) \
        * prob_ref[:TILE, :1]
    out_ref[pl.ds(0, TILE), :] = jnp.dot(act_a, wdn_a_ref[0],
                                         preferred_element_type=F32)
    xb = xg_ref[pl.ds(TILE, TILE), :]
    hg_b = jnp.dot(xb, wup_b_ref[0], preferred_element_type=F32)
    h_b, g_b = hg_b[:, :H], hg_b[:, H:]
    act_b = (0.5 * h_b * (1.0 + jax.lax.erf(h_b * c))) * (g_b + 1.0) \
        * prob_ref[TILE:, :1]
    out_ref[pl.ds(TILE, TILE), :] = jnp.dot(act_b, wdn_b_ref[0],
                                            preferred_element_type=F32)


def _grouped_gemm(te, row_tok, x, prob2d, W_up, W_down):
    grid_spec = pltpu.PrefetchScalarGridSpec(
        num_scalar_prefetch=2,
        grid=(NT2,),
        in_specs=[
            pl.BlockSpec((T, D), lambda i, te, rt: (0, 0)),
            pl.BlockSpec((2 * TILE, 128), lambda i, te, rt: (i, 0)),
            pl.BlockSpec((1, D, 2 * H), lambda i, te, rt: (te[2 * i], 0, 0)),
            pl.BlockSpec((1, H, D), lambda i, te, rt: (te[2 * i], 0, 0)),
            pl.BlockSpec((1, D, 2 * H),
                         lambda i, te, rt: (te[2 * i + 1], 0, 0)),
            pl.BlockSpec((1, H, D), lambda i, te, rt: (te[2 * i + 1], 0, 0)),
        ],
        out_specs=pl.BlockSpec((2 * TILE, D), lambda i, te, rt: (i, 0)),
        scratch_shapes=[pltpu.VMEM((2 * TILE, D), F32)],
    )
    return pl.pallas_call(
        _gemm_body,
        grid_spec=grid_spec,
        out_shape=jax.ShapeDtypeStruct((ROWS, D), F32),
        compiler_params=pltpu.CompilerParams(
            dimension_semantics=("parallel",)),
    )(te, row_tok, x, prob2d, W_up, W_down, W_up, W_down)


NC, NS, L = 2, 16, 16
NW = NC * NS
RPW = ROWS // NW
TPW = T // NW
SPW = S // NW

_SC_MESH = plsc.VectorSubcoreMesh(core_axis_name="c", subcore_axis_name="s",
                                  num_cores=NC, num_subcores=NS)


def _combine_body(ys_hbm, pos_hbm, out_hbm, idx_v, rows_v, out_v, sem):
    wid = jax.lax.axis_index("s") * NC + jax.lax.axis_index("c")
    pltpu.sync_copy(pos_hbm.at[pl.ds(wid * SPW, SPW)], idx_v)
    pltpu.async_copy(ys_hbm.at[idx_v], rows_v, sem).wait()

    def tok_body(t, _):
        def chunk_body(c, _):
            a = rows_v[2 * t, pl.ds(c * L, L)]
            b = rows_v[2 * t + 1, pl.ds(c * L, L)]
            out_v[t, pl.ds(c * L, L)] = a + b
            return 0
        return jax.lax.fori_loop(0, D // L, chunk_body, 0)

    jax.lax.fori_loop(0, TPW, tok_body, 0)
    pltpu.sync_copy(out_v, out_hbm.at[pl.ds(wid * TPW, TPW)])


_combine = functools.partial(
    pl.kernel, _combine_body,
    out_type=jax.ShapeDtypeStruct((T, D), F32),
    mesh=_SC_MESH,
    scratch_types=[
        pltpu.VMEM((SPW,), jnp.int32),
        pltpu.VMEM((SPW, D), F32),
        pltpu.VMEM((TPW, D), F32),
        pltpu.SemaphoreType.DMA,
    ],
)()


def kernel(x, expert_p, expert_idxs, W_up, W_down):
    eflat = expert_idxs.astype(jnp.int32).reshape(S, 1)
    pflat = expert_p.astype(F32).reshape(S, 1)
    pos, row_tok, prob2d, te = _routing(eflat, pflat)
    pos = pos.reshape(S)
    row_tok = row_tok.reshape(ROWS)
    te = te.reshape(NTE)

    y_sorted = _grouped_gemm(te, row_tok, x, prob2d, W_up, W_down)
    y = _combine(y_sorted, pos)
    return y


# speedup vs baseline: 1.0010x; 1.0010x over previous
import functools

import jax
import jax.numpy as jnp
from jax.experimental import pallas as pl
from jax.experimental.pallas import tpu as pltpu
from jax.experimental.pallas import tpu_sc as plsc

T, D, H, E, TOPK = 256, 1024, 512, 64, 2
S = T * TOPK
TILE = 32
NT = 78
NTE = 80
ROWS = 2560
F32 = jnp.float32


def _routing_body(eflat_ref, pflat_ref, pos_ref, row_tok_ref, prob2d_ref,
                  te_ref):
    eflat = eflat_ref[...]
    pflat = pflat_ref[...]
    iota_e = jax.lax.broadcasted_iota(jnp.int32, (1, E), 1)
    onehot = (eflat == iota_e).astype(F32)
    r_i = jax.lax.broadcasted_iota(jnp.int32, (S, S), 0)
    c_i = jax.lax.broadcasted_iota(jnp.int32, (S, S), 1)
    lts = (r_i >= c_i).astype(F32)
    cum = jnp.dot(lts, onehot, preferred_element_type=F32)
    rank = jnp.sum(onehot * (cum - 1.0), axis=1, keepdims=True)
    counts = jnp.sum(onehot, axis=0, keepdims=True)
    padded = jnp.floor((counts + (TILE - 1)) / TILE) * TILE
    r64 = jax.lax.broadcasted_iota(jnp.int32, (E, E), 0)
    c64 = jax.lax.broadcasted_iota(jnp.int32, (E, E), 1)
    mstrict = (r64 < c64).astype(F32)
    start = jnp.dot(padded, mstrict, preferred_element_type=F32, precision=jax.lax.Precision.HIGHEST)
    pos_f = jnp.sum(onehot * start, axis=1, keepdims=True) + rank
    eye = (r_i == c_i).astype(F32)
    pos_t = jnp.dot(jnp.ones((1, S), F32), eye * pos_f,
                    preferred_element_type=F32, precision=jax.lax.Precision.HIGHEST)
    rr = jax.lax.broadcasted_iota(jnp.int32, (ROWS, 1), 0).astype(F32)
    perm = (rr == pos_t).astype(F32)
    tok = (jax.lax.broadcasted_iota(jnp.int32, (S, 1), 0) // TOPK).astype(F32)
    rhs = jnp.concatenate([tok, pflat, jnp.ones((S, 1), F32)], axis=1)
    rowinfo = jnp.dot(perm, rhs, preferred_element_type=F32,
                      precision=jax.lax.Precision.HIGHEST)
    row_tok = rowinfo[:, 0:1]
    row_prob = rowinfo[:, 1:2]
    valid = rowinfo[:, 2:3]
    rr_mod = rr - jnp.floor(rr / T) * T
    row_tok = jnp.where(valid > 0.5, row_tok, rr_mod)
    ti = jax.lax.broadcasted_iota(jnp.int32, (NTE, 1), 0).astype(F32) * TILE
    te_hit = ((ti >= start) & (ti < start + padded)).astype(F32)
    tile_expert = jnp.sum(te_hit * iota_e.astype(F32), axis=1, keepdims=True)
    tile_hit = jnp.sum(te_hit, axis=1, keepdims=True)
    tile_expert = jnp.where(tile_hit > 0.5, tile_expert,
                            jnp.max(tile_expert))

    pos_ref[...] = pos_f.astype(jnp.int32)
    row_tok_ref[...] = row_tok.astype(jnp.int32)
    prob2d_ref[...] = jnp.broadcast_to(row_prob, (ROWS, 128))
    te_ref[...] = tile_expert.astype(jnp.int32)


def _routing(eflat, pflat):
    return pl.pallas_call(
        _routing_body,
        out_shape=(
            jax.ShapeDtypeStruct((S, 1), jnp.int32),
            jax.ShapeDtypeStruct((ROWS, 1), jnp.int32),
            jax.ShapeDtypeStruct((ROWS, 128), F32),
            jax.ShapeDtypeStruct((NTE, 1), jnp.int32),
        ),
    )(eflat, pflat)


NT2 = NTE // 2


def _gemm_body(te_ref, rt_ref, x_ref, prob_ref, wup_a_ref, wdn_a_ref,
               wup_b_ref, wdn_b_ref, out_ref, xg_ref):
    i = pl.program_id(0)
    for r in range(2 * TILE):
        tok = rt_ref[i * 2 * TILE + r]
        xg_ref[pl.ds(r, 1), :] = x_ref[pl.ds(tok, 1), :]
    c = 0.7071067811865476
    xa = xg_ref[pl.ds(0, TILE), :]
    hg_a = jnp.dot(xa, wup_a_ref[0], preferred_element_type=F32)
    h_a, g_a = hg_a[:, :H], hg_a[:, H:]
    act_a = (0.5 * h_a * (1.0 + jax.lax.erf(h_a * c))) * (g_a + 1.0) \
        * prob_ref[:TILE, :1]
    out_ref[pl.ds(0, TILE), :] = jnp.dot(act_a, wdn_a_ref[0],
                                         preferred_element_type=F32)
    xb = xg_ref[pl.ds(TILE, TILE), :]
    hg_b = jnp.dot(xb, wup_b_ref[0], preferred_element_type=F32)
    h_b, g_b = hg_b[:, :H], hg_b[:, H:]
    act_b = (0.5 * h_b * (1.0 + jax.lax.erf(h_b * c))) * (g_b + 1.0) \
        * prob_ref[TILE:, :1]
    out_ref[pl.ds(TILE, TILE), :] = jnp.dot(act_b, wdn_b_ref[0],
                                            preferred_element_type=F32)


def _grouped_gemm(te, row_tok, x, prob2d, W_up, W_down):
    grid_spec = pltpu.PrefetchScalarGridSpec(
        num_scalar_prefetch=2,
        grid=(NT2,),
        in_specs=[
            pl.BlockSpec((T, D), lambda i, te, rt: (0, 0)),
            pl.BlockSpec((2 * TILE, 128), lambda i, te, rt: (i, 0)),
            pl.BlockSpec((1, D, 2 * H), lambda i, te, rt: (te[2 * i], 0, 0)),
            pl.BlockSpec((1, H, D), lambda i, te, rt: (te[2 * i], 0, 0)),
            pl.BlockSpec((1, D, 2 * H),
                         lambda i, te, rt: (te[2 * i + 1], 0, 0)),
            pl.BlockSpec((1, H, D), lambda i, te, rt: (te[2 * i + 1], 0, 0)),
        ],
        out_specs=pl.BlockSpec((2 * TILE, D), lambda i, te, rt: (i, 0)),
        scratch_shapes=[pltpu.VMEM((2 * TILE, D), F32)],
    )
    return pl.pallas_call(
        _gemm_body,
        grid_spec=grid_spec,
        out_shape=jax.ShapeDtypeStruct((ROWS, D), F32),
    )(te, row_tok, x, prob2d, W_up, W_down, W_up, W_down)


NC, NS, L = 2, 16, 16
NW = NC * NS
RPW = ROWS // NW
TPW = T // NW
SPW = S // NW

_SC_MESH = plsc.VectorSubcoreMesh(core_axis_name="c", subcore_axis_name="s",
                                  num_cores=NC, num_subcores=NS)


def _combine_body(ys_hbm, pos_hbm, out_hbm, idx_v, rows_v, out_v, sem):
    wid = jax.lax.axis_index("s") * NC + jax.lax.axis_index("c")
    pltpu.sync_copy(pos_hbm.at[pl.ds(wid * SPW, SPW)], idx_v)
    pltpu.async_copy(ys_hbm.at[idx_v], rows_v, sem).wait()

    def tok_body(t, _):
        def chunk_body(c, _):
            a = rows_v[2 * t, pl.ds(c * L, L)]
            b = rows_v[2 * t + 1, pl.ds(c * L, L)]
            out_v[t, pl.ds(c * L, L)] = a + b
            return 0
        return jax.lax.fori_loop(0, D // L, chunk_body, 0)

    jax.lax.fori_loop(0, TPW, tok_body, 0)
    pltpu.sync_copy(out_v, out_hbm.at[pl.ds(wid * TPW, TPW)])


_combine = functools.partial(
    pl.kernel, _combine_body,
    out_type=jax.ShapeDtypeStruct((T, D), F32),
    mesh=_SC_MESH,
    scratch_types=[
        pltpu.VMEM((SPW,), jnp.int32),
        pltpu.VMEM((SPW, D), F32),
        pltpu.VMEM((TPW, D), F32),
        pltpu.SemaphoreType.DMA,
    ],
)()


def kernel(x, expert_p, expert_idxs, W_up, W_down):
    eflat = expert_idxs.astype(jnp.int32).reshape(S, 1)
    pflat = expert_p.astype(F32).reshape(S, 1)
    pos, row_tok, prob2d, te = _routing(eflat, pflat)
    pos = pos.reshape(S)
    row_tok = row_tok.reshape(ROWS)
    te = te.reshape(NTE)

    y_sorted = _grouped_gemm(te, row_tok, x, prob2d, W_up, W_down)
    y = _combine(y_sorted, pos)
    return y
